# baseline (speedup 1.0000x reference)
import functools

import jax
import jax.numpy as jnp
from jax import lax
from jax.experimental import pallas as pl
from jax.experimental.pallas import tpu as pltpu
from jax.experimental.pallas import tpu_sc as plsc

N = 10000
E = 320000
D = 128
DH = 64
NC, NS = 2, 16
ROWS_PER_TILE = 632
LAST_ROWS = N - 15 * ROWS_PER_TILE
C = 128
CHUNKS = E // C
NIB = 6
NRB = 4
AHEAD = 2
UNROLL = 12

_mesh = plsc.VectorSubcoreMesh(core_axis_name="c", subcore_axis_name="s")
_half = jax.ShapeDtypeStruct((N, DH), jnp.float32)

_SCRATCH = [
    pltpu.VMEM((NIB, C), jnp.int32),
    pltpu.VMEM((NIB, C), jnp.int32),
    pltpu.VMEM((NRB, C, DH), jnp.float32),
    pltpu.VMEM_SHARED((N, DH), jnp.float32),
    pltpu.SemaphoreType.DMA((NIB,)),
    pltpu.SemaphoreType.DMA((NRB,)),
    pltpu.SemaphoreType.DMA((NRB,)),
]


def _tile_slices(sid):
    rsl = pl.ds(sid * ROWS_PER_TILE, ROWS_PER_TILE)
    rsl_last = pl.ds(15 * ROWS_PER_TILE, LAST_ROWS)
    return rsl, rsl_last


def _sliced(f, sid, a_full, a_last, b_full, b_last):
    @pl.when(sid < 15)
    def _():
        f(a_full, b_full)

    @pl.when(sid == 15)
    def _():
        f(a_last, b_last)


def _run_edges(h_hbm, src_hbm, dst_hbm, acc_sh, sidx_v, didx_v, rows_v,
               isem, gsem, ssem, n_i, n_max, first, do_init):
    def istart(i, ib):
        base = (first + i) * C
        pltpu.async_copy(src_hbm.at[pl.ds(base, C)], sidx_v.at[ib],
                         isem.at[ib])
        pltpu.async_copy(dst_hbm.at[pl.ds(base, C)], didx_v.at[ib],
                         isem.at[ib])

    def iwait(ib):
        pltpu.make_async_copy(src_hbm.at[pl.ds(0, C)], sidx_v.at[ib],
                              isem.at[ib]).wait()
        pltpu.make_async_copy(dst_hbm.at[pl.ds(0, C)], didx_v.at[ib],
                              isem.at[ib]).wait()

    def gather_start(ib, b):
        pltpu.async_copy(h_hbm.at[sidx_v.at[ib]], rows_v.at[b], gsem.at[b])

    def gather_wait(b):
        pltpu.make_async_copy(h_hbm.at[sidx_v.at[0]], rows_v.at[b],
                              gsem.at[b]).wait()

    def scatter_start(ib, b):
        pltpu.async_copy(rows_v.at[b], acc_sh.at[didx_v.at[ib]],
                         ssem.at[b], add=True)

    def scatter_wait(b):
        pltpu.make_async_copy(rows_v.at[b], acc_sh.at[didx_v.at[0]],
                              ssem.at[b]).wait()

    for j in range(NIB):
        istart(j, j)
    iwait(0)
    gather_start(0, 0)

    do_init()
    plsc.subcore_barrier()

    def body(jj, carry):
        for k in range(UNROLL):
            s = 1 + jj * UNROLL + k
            b = s % NRB
            o = (s - 1) % NRB
            ib = s % NIB
            ibp = (s - 1) % NIB
            ibn = (s + AHEAD) % NIB

            @pl.when(s <= n_i - 1)
            def _():
                @pl.when(s >= NRB)
                def _():
                    scatter_wait(b)

                @pl.when(jnp.logical_and(s + AHEAD <= n_i - 1,
                                         s >= NIB - AHEAD))
                def _():
                    istart(s + AHEAD, ibn)

                iwait(ib)
                gather_start(ib, b)

            @pl.when(s <= n_i)
            def _():
                gather_wait(o)
                scatter_start(ibp, o)
        return carry

    lax.fori_loop(0, (n_max + UNROLL - 1) // UNROLL, body, 0)

    for b in range(NRB):
        scatter_wait(b)

    plsc.subcore_barrier()


FS_FULL = CHUNKS // NS
FS_REM = CHUNKS - FS_FULL * NS


@functools.partial(
    pl.kernel,
    mesh=_mesh,
    out_type=(_half, _half),
    compiler_params=pltpu.CompilerParams(use_tc_tiling_on_sc=False),
    scratch_types=_SCRATCH,
)
def _prop_fs(ta_hbm, tb_hbm, src_hbm, dst_hbm, oa_hbm, ob_hbm,
             sidx_v, didx_v, rows_v, acc_sh, isem, gsem, ssem):
    cid = lax.axis_index("c")
    sid = lax.axis_index("s")
    n_i = FS_FULL + jnp.where(sid < FS_REM, 1, 0)
    first = sid * FS_FULL + jnp.minimum(sid, FS_REM)
    rsl, rsl_last = _tile_slices(sid)

    def run(t_hbm, o_hbm):
        def do_init():
            _sliced(lambda s_, d_: pltpu.sync_copy(s_, d_), sid,
                    t_hbm.at[rsl], t_hbm.at[rsl_last],
                    acc_sh.at[rsl], acc_sh.at[rsl_last])

        _run_edges(t_hbm, src_hbm, dst_hbm, acc_sh, sidx_v, didx_v,
                   rows_v, isem, gsem, ssem, n_i, FS_FULL + 1, first,
                   do_init)

        _sliced(lambda s_, d_: pltpu.sync_copy(s_, d_), sid,
                acc_sh.at[rsl], acc_sh.at[rsl_last],
                o_hbm.at[rsl], o_hbm.at[rsl_last])

    @pl.when(cid == 0)
    def _():
        run(ta_hbm, oa_hbm)

    @pl.when(cid == 1)
    def _():
        run(tb_hbm, ob_hbm)


ES_PER_CORE = CHUNKS // NC
ES_FULL = ES_PER_CORE // NS
ES_REM = ES_PER_CORE - ES_FULL * NS


@functools.partial(
    pl.kernel,
    mesh=_mesh,
    out_type=(_half, _half),
    compiler_params=pltpu.CompilerParams(use_tc_tiling_on_sc=False),
    scratch_types=_SCRATCH,
)
def _prop_es(t_hbm, src_hbm, dst_hbm, zeros_hbm, o0_hbm, o1_hbm,
             sidx_v, didx_v, rows_v, acc_sh, isem, gsem, ssem):
    cid = lax.axis_index("c")
    sid = lax.axis_index("s")
    n_i = ES_FULL + jnp.where(sid < ES_REM, 1, 0)
    first = cid * ES_PER_CORE + sid * ES_FULL + jnp.minimum(sid, ES_REM)
    rsl, rsl_last = _tile_slices(sid)

    def do_init():
        @pl.when(cid == 0)
        def _():
            _sliced(lambda s_, d_: pltpu.sync_copy(s_, d_), sid,
                    t_hbm.at[rsl], t_hbm.at[rsl_last],
                    acc_sh.at[rsl], acc_sh.at[rsl_last])

        @pl.when(cid == 1)
        def _():
            _sliced(lambda s_, d_: pltpu.sync_copy(s_, d_), sid,
                    zeros_hbm.at[pl.ds(0, ROWS_PER_TILE)],
                    zeros_hbm.at[pl.ds(0, LAST_ROWS)],
                    acc_sh.at[rsl], acc_sh.at[rsl_last])

    _run_edges(t_hbm, src_hbm, dst_hbm, acc_sh, sidx_v, didx_v, rows_v,
               isem, gsem, ssem, n_i, ES_FULL + 1, first, do_init)

    def write_to(o_hbm):
        _sliced(lambda s_, d_: pltpu.sync_copy(s_, d_), sid,
                acc_sh.at[rsl], acc_sh.at[rsl_last],
                o_hbm.at[rsl], o_hbm.at[rsl_last])

    @pl.when(cid == 0)
    def _():
        write_to(o0_hbm)

    @pl.when(cid == 1)
    def _():
        write_to(o1_hbm)


ROW_BLK = 1000
GRID = N // ROW_BLK

_h_spec = pl.BlockSpec((ROW_BLK, DH), lambda i: (i, 0))

_SELU_ALPHA = 1.6732632423543772
_SELU_SCALE = 1.0507009873554805


def _mlp_body(qa_ref, qb_ref, w1_ref, b1_ref, w2_ref, o_ref):
    h2 = jnp.concatenate([qa_ref[...], qb_ref[...]], axis=1)
    z = jnp.dot(h2, w1_ref[...], preferred_element_type=jnp.float32)
    z = z + b1_ref[...]
    h = _SELU_SCALE * jnp.where(z > 0, z, _SELU_ALPHA * (jnp.exp(z) - 1.0))
    o_ref[...] = jnp.dot(h, w2_ref[...], preferred_element_type=jnp.float32)


def _mlp(qa, qb, W1, b1, W2):
    return pl.pallas_call(
        _mlp_body,
        grid=(GRID,),
        in_specs=[
            _h_spec, _h_spec,
            pl.BlockSpec((D, D), lambda i: (0, 0)),
            pl.BlockSpec((1, D), lambda i: (0, 0)),
            pl.BlockSpec((D, DH), lambda i: (0, 0)),
        ],
        out_specs=_h_spec,
        out_shape=_half,
    )(qa, qb, W1, b1.reshape(1, D), W2)


def _out_body(r0_ref, r1_ref, b_ref, o_ref):
    g = r0_ref[...] + r1_ref[...] + b_ref[...]
    m = jnp.max(g, axis=1, keepdims=True)
    e = g - m
    lse = jnp.log(jnp.sum(jnp.exp(e), axis=1, keepdims=True))
    o_ref[...] = e - lse


def _outp(r0, r1, b2):
    return pl.pallas_call(
        _out_body,
        grid=(GRID,),
        in_specs=[
            _h_spec, _h_spec,
            pl.BlockSpec((1, DH), lambda i: (0, 0)),
        ],
        out_specs=_h_spec,
        out_shape=_half,
    )(r0, r1, b2.reshape(1, DH))


def kernel(x, edge_index, W1, b1, W2, b2):
    src = edge_index[0].astype(jnp.int32)
    dst = edge_index[1].astype(jnp.int32)
    zeros = jnp.zeros((ROWS_PER_TILE, DH), jnp.float32)
    x_a = x[:, :DH]
    x_b = x[:, DH:]

    h1a, h1b = _prop_fs(x_a, x_b, src, dst)
    qa, qb = _prop_fs(h1a, h1b, src, dst)
    t = _mlp(qa, qb, W1, b1, W2)
    r0, r1 = _prop_es(t, src, dst, zeros)
    return _outp(r0, r1, b2)


# speedup vs baseline: 1.1848x; 1.1848x over previous
import functools

import jax
import jax.numpy as jnp
from jax import lax
from jax.experimental import pallas as pl
from jax.experimental.pallas import tpu as pltpu
from jax.experimental.pallas import tpu_sc as plsc

N = 10000
E = 320000
D = 128
DO = 64
NC, NS = 2, 16
ROWS_PER_TILE = 632
LAST_ROWS = N - 15 * ROWS_PER_TILE
C = 128
CHUNKS = E // C
CHUNKS_PER_CORE = CHUNKS // NC
FULL_PER_TILE = CHUNKS_PER_CORE // NS
REM = CHUNKS_PER_CORE - FULL_PER_TILE * NS
NIB = 4
NRB = 3
UNROLL = 12

_mesh = plsc.VectorSubcoreMesh(core_axis_name="c", subcore_axis_name="s")


def _make_prop(width, tc_tiling):
    shape = jax.ShapeDtypeStruct((N, width), jnp.float32)

    @functools.partial(
        pl.kernel,
        mesh=_mesh,
        out_type=(shape, shape),
        compiler_params=pltpu.CompilerParams(use_tc_tiling_on_sc=tc_tiling),
        scratch_types=[
            pltpu.VMEM((NIB, 2, C), jnp.int32),
            pltpu.VMEM((NRB, C, width), jnp.float32),
            pltpu.VMEM_SHARED((N, width), jnp.float32),
            pltpu.SemaphoreType.DMA((NIB,)),
            pltpu.SemaphoreType.DMA((NRB,)),
            pltpu.SemaphoreType.DMA((NRB,)),
        ],
    )
    def prop(h_hbm, eidx_hbm, zeros_hbm, o0_hbm, o1_hbm,
             idx_v, rows_v, acc_sh, isem, gsem, ssem):
        cid = lax.axis_index("c")
        sid = lax.axis_index("s")

        n_i = FULL_PER_TILE + jnp.where(sid < REM, 1, 0)
        first = (cid * CHUNKS_PER_CORE + sid * FULL_PER_TILE
                 + jnp.minimum(sid, REM))

        def istart(i, ib):
            base = (first + i) * C
            pltpu.async_copy(eidx_hbm.at[:, pl.ds(base, C)], idx_v.at[ib],
                             isem.at[ib])

        def iwait(ib):
            pltpu.make_async_copy(eidx_hbm.at[:, pl.ds(0, C)], idx_v.at[ib],
                                  isem.at[ib]).wait()

        def gather_start(ib, b):
            pltpu.async_copy(h_hbm.at[idx_v.at[ib, 0]], rows_v.at[b],
                             gsem.at[b])

        def gather_wait(b):
            pltpu.make_async_copy(h_hbm.at[idx_v.at[0, 0]], rows_v.at[b],
                                  gsem.at[b]).wait()

        def scatter_start(ib, b):
            pltpu.async_copy(rows_v.at[b], acc_sh.at[idx_v.at[ib, 1]],
                             ssem.at[b], add=True)

        def scatter_wait(b):
            pltpu.make_async_copy(rows_v.at[b], acc_sh.at[idx_v.at[0, 1]],
                                  ssem.at[b]).wait()

        for j in range(NIB):
            istart(j, j)
        iwait(0)
        gather_start(0, 0)

        rsl = pl.ds(sid * ROWS_PER_TILE, ROWS_PER_TILE)
        rsl_last = pl.ds(15 * ROWS_PER_TILE, LAST_ROWS)

        def init_write(src_full, src_last):
            @pl.when(sid < 15)
            def _():
                pltpu.sync_copy(src_full, acc_sh.at[rsl])

            @pl.when(sid == 15)
            def _():
                pltpu.sync_copy(src_last, acc_sh.at[rsl_last])

        @pl.when(cid == 0)
        def _():
            init_write(h_hbm.at[rsl], h_hbm.at[rsl_last])

        @pl.when(cid == 1)
        def _():
            init_write(zeros_hbm.at[pl.ds(0, ROWS_PER_TILE)],
                       zeros_hbm.at[pl.ds(0, LAST_ROWS)])

        plsc.subcore_barrier()

        def body(jj, carry):
            for k in range(UNROLL):
                s = 1 + jj * UNROLL + k
                b = s % NRB
                o = (s - 1) % NRB
                ib = s % NIB
                ibp = (s - 1) % NIB
                ibn = (s + 1) % NIB

                @pl.when(s <= n_i - 1)
                def _():
                    @pl.when(s >= NRB)
                    def _():
                        scatter_wait(b)

                    @pl.when(jnp.logical_and(s + 1 <= n_i - 1,
                                             s >= NIB - 1))
                    def _():
                        istart(s + 1, ibn)

                    iwait(ib)
                    gather_start(ib, b)

                @pl.when(s <= n_i)
                def _():
                    gather_wait(o)
                    scatter_start(ibp, o)
            return carry

        lax.fori_loop(0, (FULL_PER_TILE + 1 + UNROLL - 1) // UNROLL, body, 0)

        for b in range(NRB):
            scatter_wait(b)

        plsc.subcore_barrier()

        def write_to(o_hbm):
            @pl.when(sid < 15)
            def _():
                pltpu.sync_copy(acc_sh.at[rsl], o_hbm.at[rsl])

            @pl.when(sid == 15)
            def _():
                pltpu.sync_copy(acc_sh.at[rsl_last], o_hbm.at[rsl_last])

        @pl.when(cid == 0)
        def _():
            write_to(o0_hbm)

        @pl.when(cid == 1)
        def _():
            write_to(o1_hbm)

    return prop


_prop = _make_prop(D, True)
_prop_out = _make_prop(DO, False)


ROW_BLK = 1000
GRID = N // ROW_BLK

_blk_spec = pl.BlockSpec((ROW_BLK, D), lambda i: (i, 0))
_out_spec = pl.BlockSpec((ROW_BLK, DO), lambda i: (i, 0))
_full = jax.ShapeDtypeStruct((N, D), jnp.float32)
_half = jax.ShapeDtypeStruct((N, DO), jnp.float32)

_SELU_ALPHA = 1.6732632423543772
_SELU_SCALE = 1.0507009873554805


def _add2_body(p0_ref, p1_ref, o_ref):
    o_ref[...] = p0_ref[...] + p1_ref[...]


def _add2(p0, p1):
    return pl.pallas_call(
        _add2_body,
        grid=(GRID,),
        in_specs=[_blk_spec, _blk_spec],
        out_specs=_blk_spec,
        out_shape=_full,
    )(p0, p1)


def _mlp_body(q0_ref, q1_ref, w1_ref, b1_ref, w2_ref, o_ref):
    h2 = q0_ref[...] + q1_ref[...]
    z = jnp.dot(h2, w1_ref[...], preferred_element_type=jnp.float32)
    z = z + b1_ref[...]
    h = _SELU_SCALE * jnp.where(z > 0, z, _SELU_ALPHA * (jnp.exp(z) - 1.0))
    o_ref[...] = jnp.dot(h, w2_ref[...], preferred_element_type=jnp.float32)


def _mlp(q0, q1, W1, b1, W2):
    return pl.pallas_call(
        _mlp_body,
        grid=(GRID,),
        in_specs=[
            _blk_spec, _blk_spec,
            pl.BlockSpec((D, D), lambda i: (0, 0)),
            pl.BlockSpec((1, D), lambda i: (0, 0)),
            pl.BlockSpec((D, DO), lambda i: (0, 0)),
        ],
        out_specs=_out_spec,
        out_shape=_half,
    )(q0, q1, W1, b1.reshape(1, D), W2)


def _out_body(r0_ref, r1_ref, b_ref, o_ref):
    g = r0_ref[...] + r1_ref[...] + b_ref[...]
    m = jnp.max(g, axis=1, keepdims=True)
    e = g - m
    lse = jnp.log(jnp.sum(jnp.exp(e), axis=1, keepdims=True))
    o_ref[...] = e - lse


def _outp(r0, r1, b2):
    return pl.pallas_call(
        _out_body,
        grid=(GRID,),
        in_specs=[
            _out_spec, _out_spec,
            pl.BlockSpec((1, DO), lambda i: (0, 0)),
        ],
        out_specs=_out_spec,
        out_shape=_half,
    )(r0, r1, b2.reshape(1, DO))


def kernel(x, edge_index, W1, b1, W2, b2):
    eidx = edge_index.astype(jnp.int32)
    zeros = jnp.zeros((ROWS_PER_TILE, D), jnp.float32)
    zeros_o = jnp.zeros((ROWS_PER_TILE, DO), jnp.float32)

    p0, p1 = _prop(x, eidx, zeros)
    h1 = _add2(p0, p1)
    q0, q1 = _prop(h1, eidx, zeros)
    t = _mlp(q0, q1, W1, b1, W2)
    r0, r1 = _prop_out(t, eidx, zeros_o)
    return _outp(r0, r1, b2)


# speedup vs baseline: 1.1856x; 1.0006x over previous
import functools

import jax
import jax.numpy as jnp
from jax import lax
from jax.experimental import pallas as pl
from jax.experimental.pallas import tpu as pltpu
from jax.experimental.pallas import tpu_sc as plsc

N = 10000
E = 320000
D = 128
DO = 64
NC, NS = 2, 16
ROWS_PER_TILE = 632
LAST_ROWS = N - 15 * ROWS_PER_TILE
C = 128
CHUNKS = E // C
CHUNKS_PER_CORE = CHUNKS // NC
FULL_PER_TILE = CHUNKS_PER_CORE // NS
REM = CHUNKS_PER_CORE - FULL_PER_TILE * NS
UNROLL = 12

_mesh = plsc.VectorSubcoreMesh(core_axis_name="c", subcore_axis_name="s")


def _make_prop(width, tc_tiling, NRB, NIB):
    shape = jax.ShapeDtypeStruct((N, width), jnp.float32)

    @functools.partial(
        pl.kernel,
        mesh=_mesh,
        out_type=(shape, shape),
        compiler_params=pltpu.CompilerParams(use_tc_tiling_on_sc=tc_tiling),
        scratch_types=[
            pltpu.VMEM((NIB, 2, C), jnp.int32),
            pltpu.VMEM((NRB, C, width), jnp.float32),
            pltpu.VMEM_SHARED((N, width), jnp.float32),
            pltpu.SemaphoreType.DMA((NIB,)),
            pltpu.SemaphoreType.DMA((NRB,)),
            pltpu.SemaphoreType.DMA((NRB,)),
        ],
    )
    def prop(h_hbm, eidx_hbm, zeros_hbm, o0_hbm, o1_hbm,
             idx_v, rows_v, acc_sh, isem, gsem, ssem):
        cid = lax.axis_index("c")
        sid = lax.axis_index("s")

        n_i = FULL_PER_TILE + jnp.where(sid < REM, 1, 0)
        first = (cid * CHUNKS_PER_CORE + sid * FULL_PER_TILE
                 + jnp.minimum(sid, REM))

        def istart(i, ib):
            base = (first + i) * C
            pltpu.async_copy(eidx_hbm.at[:, pl.ds(base, C)], idx_v.at[ib],
                             isem.at[ib])

        def iwait(ib):
            pltpu.make_async_copy(eidx_hbm.at[:, pl.ds(0, C)], idx_v.at[ib],
                                  isem.at[ib]).wait()

        def gather_start(ib, b):
            pltpu.async_copy(h_hbm.at[idx_v.at[ib, 0]], rows_v.at[b],
                             gsem.at[b])

        def gather_wait(b):
            pltpu.make_async_copy(h_hbm.at[idx_v.at[0, 0]], rows_v.at[b],
                                  gsem.at[b]).wait()

        def scatter_start(ib, b):
            pltpu.async_copy(rows_v.at[b], acc_sh.at[idx_v.at[ib, 1]],
                             ssem.at[b], add=True)

        def scatter_wait(b):
            pltpu.make_async_copy(rows_v.at[b], acc_sh.at[idx_v.at[0, 1]],
                                  ssem.at[b]).wait()

        for j in range(NIB):
            istart(j, j)
        iwait(0)
        gather_start(0, 0)

        rsl = pl.ds(sid * ROWS_PER_TILE, ROWS_PER_TILE)
        rsl_last = pl.ds(15 * ROWS_PER_TILE, LAST_ROWS)

        def init_write(src_full, src_last):
            @pl.when(sid < 15)
            def _():
                pltpu.sync_copy(src_full, acc_sh.at[rsl])

            @pl.when(sid == 15)
            def _():
                pltpu.sync_copy(src_last, acc_sh.at[rsl_last])

        @pl.when(cid == 0)
        def _():
            init_write(h_hbm.at[rsl], h_hbm.at[rsl_last])

        @pl.when(cid == 1)
        def _():
            init_write(zeros_hbm.at[pl.ds(0, ROWS_PER_TILE)],
                       zeros_hbm.at[pl.ds(0, LAST_ROWS)])

        plsc.subcore_barrier()

        def body(jj, carry):
            for k in range(UNROLL):
                s = 1 + jj * UNROLL + k
                b = s % NRB
                o = (s - 1) % NRB
                ib = s % NIB
                ibp = (s - 1) % NIB
                ibn = (s + 1) % NIB

                @pl.when(s <= n_i - 1)
                def _():
                    @pl.when(s >= NRB)
                    def _():
                        scatter_wait(b)

                    @pl.when(jnp.logical_and(s + 1 <= n_i - 1,
                                             s >= NIB - 1))
                    def _():
                        istart(s + 1, ibn)

                    iwait(ib)
                    gather_start(ib, b)

                @pl.when(s <= n_i)
                def _():
                    gather_wait(o)
                    scatter_start(ibp, o)
            return carry

        lax.fori_loop(0, (FULL_PER_TILE + 1 + UNROLL - 1) // UNROLL, body, 0)

        for b in range(NRB):
            scatter_wait(b)

        plsc.subcore_barrier()

        def write_to(o_hbm):
            @pl.when(sid < 15)
            def _():
                pltpu.sync_copy(acc_sh.at[rsl], o_hbm.at[rsl])

            @pl.when(sid == 15)
            def _():
                pltpu.sync_copy(acc_sh.at[rsl_last], o_hbm.at[rsl_last])

        @pl.when(cid == 0)
        def _():
            write_to(o0_hbm)

        @pl.when(cid == 1)
        def _():
            write_to(o1_hbm)

    return prop


_prop = _make_prop(D, True, 3, 4)
_prop_out = _make_prop(DO, False, 4, 6)


ROW_BLK = 1000
GRID = N // ROW_BLK

_blk_spec = pl.BlockSpec((ROW_BLK, D), lambda i: (i, 0))
_out_spec = pl.BlockSpec((ROW_BLK, DO), lambda i: (i, 0))
_full = jax.ShapeDtypeStruct((N, D), jnp.float32)
_half = jax.ShapeDtypeStruct((N, DO), jnp.float32)

_SELU_ALPHA = 1.6732632423543772
_SELU_SCALE = 1.0507009873554805


def _add2_body(p0_ref, p1_ref, o_ref):
    o_ref[...] = p0_ref[...] + p1_ref[...]


def _add2(p0, p1):
    return pl.pallas_call(
        _add2_body,
        grid=(GRID,),
        in_specs=[_blk_spec, _blk_spec],
        out_specs=_blk_spec,
        out_shape=_full,
    )(p0, p1)


def _mlp_body(q0_ref, q1_ref, w1_ref, b1_ref, w2_ref, o_ref):
    h2 = q0_ref[...] + q1_ref[...]
    z = jnp.dot(h2, w1_ref[...], preferred_element_type=jnp.float32)
    z = z + b1_ref[...]
    h = _SELU_SCALE * jnp.where(z > 0, z, _SELU_ALPHA * (jnp.exp(z) - 1.0))
    o_ref[...] = jnp.dot(h, w2_ref[...], preferred_element_type=jnp.float32)


def _mlp(q0, q1, W1, b1, W2):
    return pl.pallas_call(
        _mlp_body,
        grid=(GRID,),
        in_specs=[
            _blk_spec, _blk_spec,
            pl.BlockSpec((D, D), lambda i: (0, 0)),
            pl.BlockSpec((1, D), lambda i: (0, 0)),
            pl.BlockSpec((D, DO), lambda i: (0, 0)),
        ],
        out_specs=_out_spec,
        out_shape=_half,
    )(q0, q1, W1, b1.reshape(1, D), W2)


def _out_body(r0_ref, r1_ref, b_ref, o_ref):
    g = r0_ref[...] + r1_ref[...] + b_ref[...]
    m = jnp.max(g, axis=1, keepdims=True)
    e = g - m
    lse = jnp.log(jnp.sum(jnp.exp(e), axis=1, keepdims=True))
    o_ref[...] = e - lse


def _outp(r0, r1, b2):
    return pl.pallas_call(
        _out_body,
        grid=(GRID,),
        in_specs=[
            _out_spec, _out_spec,
            pl.BlockSpec((1, DO), lambda i: (0, 0)),
        ],
        out_specs=_out_spec,
        out_shape=_half,
    )(r0, r1, b2.reshape(1, DO))


def kernel(x, edge_index, W1, b1, W2, b2):
    eidx = edge_index.astype(jnp.int32)
    zeros = jnp.zeros((ROWS_PER_TILE, D), jnp.float32)
    zeros_o = jnp.zeros((ROWS_PER_TILE, DO), jnp.float32)

    p0, p1 = _prop(x, eidx, zeros)
    h1 = _add2(p0, p1)
    q0, q1 = _prop(h1, eidx, zeros)
    t = _mlp(q0, q1, W1, b1, W2)
    r0, r1 = _prop_out(t, eidx, zeros_o)
    return _outp(r0, r1, b2)
